# baseline (speedup 1.0000x reference)
import numpy as np

import jax
import jax.numpy as jnp
from jax.experimental import pallas as pl
from jax.experimental.pallas import tpu as pltpu

N = 4096
K = 2048
BM = 512
FB = 512
NB = N // BM
SW = 32
MARGIN = 0.3
BIG = 1e30


def _lane_tree(m, op):
    w = m.shape[1] // 2
    while w >= 128:
        m = op(m[:, 0:w], m[:, w:2 * w])
        w //= 2
    return m


def _sub_tree(m, op):
    h = m.shape[0] // 2
    while h >= 8:
        m = op(m[0:h, :], m[h:2 * h, :])
        h //= 2
    return m


def _fused_kernel(ph_arr, fb_arr, i_arr, j_arr,
                  feat_ref, lr_ref, lc_ref, tr_ref, tc_ref,
                  orap_ref, oran_ref, orapc_ref, oranc_ref,
                  ocap_ref, ocan_ref, ocapc_ref, ocanc_ref,
                  av, rap, ran, rapc, ranc, cap, can, capc, canc):
    t = pl.program_id(0)
    nt = pl.num_programs(0)

    @pl.when(t == 0)
    def _init():
        rap[...] = jnp.full((N, 1), -BIG, jnp.bfloat16)
        rapc[...] = jnp.full((N, 1), -BIG, jnp.bfloat16)
        ran[...] = jnp.full((N, 1), BIG, jnp.bfloat16)
        ranc[...] = jnp.full((N, 1), BIG, jnp.bfloat16)
        cap[...] = jnp.full((8 * NB, BM), -BIG, jnp.float32)
        capc[...] = jnp.full((8 * NB, BM), -BIG, jnp.float32)
        can[...] = jnp.full((8 * NB, BM), BIG, jnp.float32)
        canc[...] = jnp.full((8 * NB, BM), BIG, jnp.float32)

    @pl.when(ph_arr[t] == 0)
    def _norm_phase():
        x = feat_ref[...]
        l1 = jnp.clip(jnp.sum(jnp.abs(x), axis=1, keepdims=True), 1e-12, None)
        av[pl.ds(fb_arr[t] * FB, FB), :] = (x / l1).astype(jnp.bfloat16)

    @pl.when(ph_arr[t] == 1)
    def _mine_phase():
        i = i_arr[t]
        j = j_arr[t]
        sim = jax.lax.dot_general(
            av[pl.ds(i * BM, BM), :], av[pl.ds(j * BM, BM), :],
            (((1,), (1,)), ((), ())),
            preferred_element_type=jnp.float32).astype(jnp.bfloat16)

        lr = lr_ref[...]
        lc = lc_ref[...]
        tr = tr_ref[...]
        tc = tc_ref[...]

        capv = jnp.full((8, BM), -BIG, jnp.bfloat16)
        capcv = jnp.full((8, BM), -BIG, jnp.bfloat16)
        canv = jnp.full((8, BM), BIG, jnp.bfloat16)
        cancv = jnp.full((8, BM), BIG, jnp.bfloat16)
        rap_parts, ran_parts, rapc_parts, ranc_parts = [], [], [], []

        for s in range(BM // SW):
            rows = slice(s * SW, (s + 1) * SW)
            sim_s = sim[rows, :]
            l_eq = lr[rows, :] == lc
            t_eq = tr[rows, :] == tc
            u_max = jnp.where(l_eq, sim_s, -BIG)
            u_min = jnp.where(l_eq, BIG, sim_s)
            m_ap = jnp.where(t_eq, u_max, -BIG)
            m_apc = jnp.where(t_eq, -BIG, u_max)
            m_an = jnp.where(t_eq, u_min, BIG)
            m_anc = jnp.where(t_eq, BIG, u_min)
            rap_parts.append(jnp.max(_lane_tree(m_ap, jnp.maximum), axis=1, keepdims=True))
            ran_parts.append(jnp.min(_lane_tree(m_an, jnp.minimum), axis=1, keepdims=True))
            rapc_parts.append(jnp.max(_lane_tree(m_apc, jnp.maximum), axis=1, keepdims=True))
            ranc_parts.append(jnp.min(_lane_tree(m_anc, jnp.minimum), axis=1, keepdims=True))
            capv = jnp.maximum(capv, _sub_tree(m_ap, jnp.maximum))
            canv = jnp.minimum(canv, _sub_tree(m_an, jnp.minimum))
            capcv = jnp.maximum(capcv, _sub_tree(m_apc, jnp.maximum))
            cancv = jnp.minimum(cancv, _sub_tree(m_anc, jnp.minimum))

        rsl = (pl.ds(i * BM, BM), slice(None))
        csl = (pl.ds(j * 8, 8), slice(None))
        rap[rsl] = jnp.maximum(rap[rsl], jnp.concatenate(rap_parts, axis=0))
        ran[rsl] = jnp.minimum(ran[rsl], jnp.concatenate(ran_parts, axis=0))
        rapc[rsl] = jnp.maximum(rapc[rsl], jnp.concatenate(rapc_parts, axis=0))
        ranc[rsl] = jnp.minimum(ranc[rsl], jnp.concatenate(ranc_parts, axis=0))
        cap[csl] = jnp.maximum(cap[csl], capv.astype(jnp.float32))
        can[csl] = jnp.minimum(can[csl], canv.astype(jnp.float32))
        capc[csl] = jnp.maximum(capc[csl], capcv.astype(jnp.float32))
        canc[csl] = jnp.minimum(canc[csl], cancv.astype(jnp.float32))

    @pl.when(t == nt - 1)
    def _emit():
        orap_ref[...] = rap[...]
        oran_ref[...] = ran[...]
        orapc_ref[...] = rapc[...]
        oranc_ref[...] = ranc[...]
        ocap_ref[...] = cap[...]
        ocan_ref[...] = can[...]
        ocapc_ref[...] = capc[...]
        ocanc_ref[...] = canc[...]


def _dist_max(row_ref, colp_ref):
    m = jnp.maximum(row_ref[...].astype(jnp.float32),
                    jnp.max(colp_ref[...], axis=1))
    return jnp.where(m > -1e29, jnp.sqrt(jnp.clip(m, 1e-12, None)), m)


def _dist_min(row_ref, colp_ref):
    m = jnp.minimum(row_ref[...].astype(jnp.float32),
                    jnp.min(colp_ref[...], axis=1))
    return jnp.where(m < 1e29, jnp.sqrt(jnp.clip(m, 1e-12, None)), m)


def _finish_kernel(rap_ref, ran_ref, rapc_ref, ranc_ref,
                   cap_ref, can_ref, capc_ref, canc_ref, o_ref):
    ap = _dist_max(rap_ref, cap_ref)
    an = _dist_min(ran_ref, can_ref)
    apc = _dist_max(rapc_ref, capc_ref)
    anc = _dist_min(ranc_ref, canc_ref)
    loss_same = jnp.maximum(ap - an + MARGIN, 0.0)
    loss_cross = jnp.maximum(apc - anc + MARGIN, 0.0)
    total = jnp.sum(loss_same, keepdims=True) + jnp.sum(loss_cross, keepdims=True)
    o_ref[...] = total.reshape(1, 1) * (1.0 / N)


def _schedule():
    pairs = [(i, j) for i in range(NB) for j in range(i, NB)]
    ph, fb, ii, jj = [], [], [], []
    normed = 0

    def _need(upto):
        nonlocal normed
        while normed < upto:
            ph.append(0)
            fb.append(normed)
            ii.append(0)
            jj.append(0)
            normed += 1

    for (i, j) in pairs:
        _need((max(i, j) + 1) * (BM // FB))
        ph.append(1)
        fb.append(max(normed - 1, 0))
        ii.append(i)
        jj.append(j)
    return ph, fb, ii, jj


def kernel(feat, labels, tags):
    lab_r = labels.reshape(N, 1)
    lab_c = labels.reshape(1, N)
    tag_r = tags.reshape(N, 1)
    tag_c = tags.reshape(1, N)

    ph, fb, ii, jj = _schedule()
    ph_arr = jnp.asarray(np.array(ph, np.int32))
    fb_arr = jnp.asarray(np.array(fb, np.int32))
    i_arr = jnp.asarray(np.array(ii, np.int32))
    j_arr = jnp.asarray(np.array(jj, np.int32))
    nsteps = len(ph)

    vec_r = jax.ShapeDtypeStruct((N, 1), jnp.bfloat16)
    vec_c = jax.ShapeDtypeStruct((8 * NB, BM), jnp.float32)

    grid_spec = pltpu.PrefetchScalarGridSpec(
        num_scalar_prefetch=4,
        grid=(nsteps,),
        in_specs=[
            pl.BlockSpec((FB, K), lambda t, ph, fbv, ia, ja: (fbv[t], 0)),
            pl.BlockSpec((BM, 1), lambda t, ph, fbv, ia, ja: (ia[t], 0)),
            pl.BlockSpec((1, BM), lambda t, ph, fbv, ia, ja: (0, ja[t])),
            pl.BlockSpec((BM, 1), lambda t, ph, fbv, ia, ja: (ia[t], 0)),
            pl.BlockSpec((1, BM), lambda t, ph, fbv, ia, ja: (0, ja[t])),
        ],
        out_specs=[
            pl.BlockSpec((N, 1), lambda t, ph, fbv, ia, ja: (0, 0)),
            pl.BlockSpec((N, 1), lambda t, ph, fbv, ia, ja: (0, 0)),
            pl.BlockSpec((N, 1), lambda t, ph, fbv, ia, ja: (0, 0)),
            pl.BlockSpec((N, 1), lambda t, ph, fbv, ia, ja: (0, 0)),
            pl.BlockSpec((8 * NB, BM), lambda t, ph, fbv, ia, ja: (0, 0)),
            pl.BlockSpec((8 * NB, BM), lambda t, ph, fbv, ia, ja: (0, 0)),
            pl.BlockSpec((8 * NB, BM), lambda t, ph, fbv, ia, ja: (0, 0)),
            pl.BlockSpec((8 * NB, BM), lambda t, ph, fbv, ia, ja: (0, 0)),
        ],
        scratch_shapes=[
            pltpu.VMEM((N, K), jnp.bfloat16),
            pltpu.VMEM((N, 1), jnp.bfloat16), pltpu.VMEM((N, 1), jnp.bfloat16),
            pltpu.VMEM((N, 1), jnp.bfloat16), pltpu.VMEM((N, 1), jnp.bfloat16),
            pltpu.VMEM((8 * NB, BM), jnp.float32), pltpu.VMEM((8 * NB, BM), jnp.float32),
            pltpu.VMEM((8 * NB, BM), jnp.float32), pltpu.VMEM((8 * NB, BM), jnp.float32),
        ],
    )
    outs = pl.pallas_call(
        _fused_kernel,
        grid_spec=grid_spec,
        out_shape=[vec_r, vec_r, vec_r, vec_r, vec_c, vec_c, vec_c, vec_c],
        compiler_params=pltpu.CompilerParams(
            dimension_semantics=("arbitrary",)),
    )(ph_arr, fb_arr, i_arr, j_arr, feat, lab_r, lab_c, tag_r, tag_c)

    r_ap, r_an, r_apc, r_anc, c_ap, c_an, c_apc, c_anc = outs
    rT = [x.reshape(N // 128, 128) for x in (r_ap, r_an, r_apc, r_anc)]
    cT = [x.reshape(NB, 8, BM // 128, 128).transpose(0, 2, 1, 3).reshape(N // 128, 8, 128)
          for x in (c_ap, c_an, c_apc, c_anc)]

    loss = pl.pallas_call(
        _finish_kernel,
        in_specs=[pl.BlockSpec((N // 128, 128), lambda: (0, 0))] * 4
        + [pl.BlockSpec((N // 128, 8, 128), lambda: (0, 0, 0))] * 4,
        out_specs=pl.BlockSpec((1, 1), lambda: (0, 0)),
        out_shape=jax.ShapeDtypeStruct((1, 1), jnp.float32),
    )(*rT, *cT)
    return loss.reshape(())


# speedup vs baseline: 1.1167x; 1.1167x over previous
import numpy as np

import jax
import jax.numpy as jnp
from jax.experimental import pallas as pl
from jax.experimental.pallas import tpu as pltpu

N = 4096
K = 2048
BM = 1024
FB = 512
NB = N // BM
SW = 32
MARGIN = 0.3
BIG = 1e30


def _lane_tree(m, op):
    w = m.shape[1] // 2
    while w >= 128:
        m = op(m[:, 0:w], m[:, w:2 * w])
        w //= 2
    return m


def _sub_tree(m, op):
    h = m.shape[0] // 2
    while h >= 8:
        m = op(m[0:h, :], m[h:2 * h, :])
        h //= 2
    return m


def _fused_kernel(ph_arr, fb_arr, i_arr, j_arr,
                  feat_ref, lr_ref, lc_ref, tr_ref, tc_ref,
                  orap_ref, oran_ref, orapc_ref, oranc_ref,
                  ocap_ref, ocan_ref, ocapc_ref, ocanc_ref,
                  av, rap, ran, rapc, ranc, cap, can, capc, canc):
    t = pl.program_id(0)
    nt = pl.num_programs(0)

    @pl.when(t == 0)
    def _init():
        rap[...] = jnp.full((N, 1), -BIG, jnp.bfloat16)
        rapc[...] = jnp.full((N, 1), -BIG, jnp.bfloat16)
        ran[...] = jnp.full((N, 1), BIG, jnp.bfloat16)
        ranc[...] = jnp.full((N, 1), BIG, jnp.bfloat16)
        cap[...] = jnp.full((8 * NB, BM), -BIG, jnp.float32)
        capc[...] = jnp.full((8 * NB, BM), -BIG, jnp.float32)
        can[...] = jnp.full((8 * NB, BM), BIG, jnp.float32)
        canc[...] = jnp.full((8 * NB, BM), BIG, jnp.float32)

    @pl.when(ph_arr[t] == 0)
    def _norm_phase():
        x = feat_ref[...]
        l1 = jnp.clip(jnp.sum(jnp.abs(x), axis=1, keepdims=True), 1e-12, None)
        av[pl.ds(fb_arr[t] * FB, FB), :] = (x / l1).astype(jnp.bfloat16)

    @pl.when(ph_arr[t] == 1)
    def _mine_phase():
        i = i_arr[t]
        j = j_arr[t]
        sim = jax.lax.dot_general(
            av[pl.ds(i * BM, BM), :], av[pl.ds(j * BM, BM), :],
            (((1,), (1,)), ((), ())),
            preferred_element_type=jnp.float32).astype(jnp.bfloat16)

        lr = lr_ref[...]
        lc = lc_ref[...]
        tr = tr_ref[...]
        tc = tc_ref[...]

        capv = jnp.full((8, BM), -BIG, jnp.bfloat16)
        capcv = jnp.full((8, BM), -BIG, jnp.bfloat16)
        canv = jnp.full((8, BM), BIG, jnp.bfloat16)
        cancv = jnp.full((8, BM), BIG, jnp.bfloat16)
        rap_parts, ran_parts, rapc_parts, ranc_parts = [], [], [], []

        for s in range(BM // SW):
            rows = slice(s * SW, (s + 1) * SW)
            sim_s = sim[rows, :]
            l_eq = lr[rows, :] == lc
            t_eq = tr[rows, :] == tc
            u_max = jnp.where(l_eq, sim_s, -BIG)
            u_min = jnp.where(l_eq, BIG, sim_s)
            m_ap = jnp.where(t_eq, u_max, -BIG)
            m_apc = jnp.where(t_eq, -BIG, u_max)
            m_an = jnp.where(t_eq, u_min, BIG)
            m_anc = jnp.where(t_eq, BIG, u_min)
            rap_parts.append(jnp.max(_lane_tree(m_ap, jnp.maximum), axis=1, keepdims=True))
            ran_parts.append(jnp.min(_lane_tree(m_an, jnp.minimum), axis=1, keepdims=True))
            rapc_parts.append(jnp.max(_lane_tree(m_apc, jnp.maximum), axis=1, keepdims=True))
            ranc_parts.append(jnp.min(_lane_tree(m_anc, jnp.minimum), axis=1, keepdims=True))
            capv = jnp.maximum(capv, _sub_tree(m_ap, jnp.maximum))
            canv = jnp.minimum(canv, _sub_tree(m_an, jnp.minimum))
            capcv = jnp.maximum(capcv, _sub_tree(m_apc, jnp.maximum))
            cancv = jnp.minimum(cancv, _sub_tree(m_anc, jnp.minimum))

        rsl = (pl.ds(i * BM, BM), slice(None))
        csl = (pl.ds(j * 8, 8), slice(None))
        rap[rsl] = jnp.maximum(rap[rsl], jnp.concatenate(rap_parts, axis=0))
        ran[rsl] = jnp.minimum(ran[rsl], jnp.concatenate(ran_parts, axis=0))
        rapc[rsl] = jnp.maximum(rapc[rsl], jnp.concatenate(rapc_parts, axis=0))
        ranc[rsl] = jnp.minimum(ranc[rsl], jnp.concatenate(ranc_parts, axis=0))
        cap[csl] = jnp.maximum(cap[csl], capv.astype(jnp.float32))
        can[csl] = jnp.minimum(can[csl], canv.astype(jnp.float32))
        capc[csl] = jnp.maximum(capc[csl], capcv.astype(jnp.float32))
        canc[csl] = jnp.minimum(canc[csl], cancv.astype(jnp.float32))

    @pl.when(t == nt - 1)
    def _emit():
        orap_ref[...] = rap[...]
        oran_ref[...] = ran[...]
        orapc_ref[...] = rapc[...]
        oranc_ref[...] = ranc[...]
        ocap_ref[...] = cap[...]
        ocan_ref[...] = can[...]
        ocapc_ref[...] = capc[...]
        ocanc_ref[...] = canc[...]


def _dist_max(row_ref, colp_ref):
    m = jnp.maximum(row_ref[...].astype(jnp.float32),
                    jnp.max(colp_ref[...], axis=1))
    return jnp.where(m > -1e29, jnp.sqrt(jnp.clip(m, 1e-12, None)), m)


def _dist_min(row_ref, colp_ref):
    m = jnp.minimum(row_ref[...].astype(jnp.float32),
                    jnp.min(colp_ref[...], axis=1))
    return jnp.where(m < 1e29, jnp.sqrt(jnp.clip(m, 1e-12, None)), m)


def _finish_kernel(rap_ref, ran_ref, rapc_ref, ranc_ref,
                   cap_ref, can_ref, capc_ref, canc_ref, o_ref):
    ap = _dist_max(rap_ref, cap_ref)
    an = _dist_min(ran_ref, can_ref)
    apc = _dist_max(rapc_ref, capc_ref)
    anc = _dist_min(ranc_ref, canc_ref)
    loss_same = jnp.maximum(ap - an + MARGIN, 0.0)
    loss_cross = jnp.maximum(apc - anc + MARGIN, 0.0)
    total = jnp.sum(loss_same, keepdims=True) + jnp.sum(loss_cross, keepdims=True)
    o_ref[...] = total.reshape(1, 1) * (1.0 / N)


def _schedule():
    pairs = [(i, j) for i in range(NB) for j in range(i, NB)]
    ph, fb, ii, jj = [], [], [], []
    normed = 0

    def _need(upto):
        nonlocal normed
        while normed < upto:
            ph.append(0)
            fb.append(normed)
            ii.append(0)
            jj.append(0)
            normed += 1

    for (i, j) in pairs:
        _need((max(i, j) + 1) * (BM // FB))
        ph.append(1)
        fb.append(max(normed - 1, 0))
        ii.append(i)
        jj.append(j)
    return ph, fb, ii, jj


def kernel(feat, labels, tags):
    lab_r = labels.reshape(N, 1)
    lab_c = labels.reshape(1, N)
    tag_r = tags.reshape(N, 1)
    tag_c = tags.reshape(1, N)

    ph, fb, ii, jj = _schedule()
    ph_arr = jnp.asarray(np.array(ph, np.int32))
    fb_arr = jnp.asarray(np.array(fb, np.int32))
    i_arr = jnp.asarray(np.array(ii, np.int32))
    j_arr = jnp.asarray(np.array(jj, np.int32))
    nsteps = len(ph)

    vec_r = jax.ShapeDtypeStruct((N, 1), jnp.bfloat16)
    vec_c = jax.ShapeDtypeStruct((8 * NB, BM), jnp.float32)

    grid_spec = pltpu.PrefetchScalarGridSpec(
        num_scalar_prefetch=4,
        grid=(nsteps,),
        in_specs=[
            pl.BlockSpec((FB, K), lambda t, ph, fbv, ia, ja: (fbv[t], 0)),
            pl.BlockSpec((BM, 1), lambda t, ph, fbv, ia, ja: (ia[t], 0)),
            pl.BlockSpec((1, BM), lambda t, ph, fbv, ia, ja: (0, ja[t])),
            pl.BlockSpec((BM, 1), lambda t, ph, fbv, ia, ja: (ia[t], 0)),
            pl.BlockSpec((1, BM), lambda t, ph, fbv, ia, ja: (0, ja[t])),
        ],
        out_specs=[
            pl.BlockSpec((N, 1), lambda t, ph, fbv, ia, ja: (0, 0)),
            pl.BlockSpec((N, 1), lambda t, ph, fbv, ia, ja: (0, 0)),
            pl.BlockSpec((N, 1), lambda t, ph, fbv, ia, ja: (0, 0)),
            pl.BlockSpec((N, 1), lambda t, ph, fbv, ia, ja: (0, 0)),
            pl.BlockSpec((8 * NB, BM), lambda t, ph, fbv, ia, ja: (0, 0)),
            pl.BlockSpec((8 * NB, BM), lambda t, ph, fbv, ia, ja: (0, 0)),
            pl.BlockSpec((8 * NB, BM), lambda t, ph, fbv, ia, ja: (0, 0)),
            pl.BlockSpec((8 * NB, BM), lambda t, ph, fbv, ia, ja: (0, 0)),
        ],
        scratch_shapes=[
            pltpu.VMEM((N, K), jnp.bfloat16),
            pltpu.VMEM((N, 1), jnp.bfloat16), pltpu.VMEM((N, 1), jnp.bfloat16),
            pltpu.VMEM((N, 1), jnp.bfloat16), pltpu.VMEM((N, 1), jnp.bfloat16),
            pltpu.VMEM((8 * NB, BM), jnp.float32), pltpu.VMEM((8 * NB, BM), jnp.float32),
            pltpu.VMEM((8 * NB, BM), jnp.float32), pltpu.VMEM((8 * NB, BM), jnp.float32),
        ],
    )
    outs = pl.pallas_call(
        _fused_kernel,
        grid_spec=grid_spec,
        out_shape=[vec_r, vec_r, vec_r, vec_r, vec_c, vec_c, vec_c, vec_c],
        compiler_params=pltpu.CompilerParams(
            dimension_semantics=("arbitrary",)),
    )(ph_arr, fb_arr, i_arr, j_arr, feat, lab_r, lab_c, tag_r, tag_c)

    r_ap, r_an, r_apc, r_anc, c_ap, c_an, c_apc, c_anc = outs
    rT = [x.reshape(N // 128, 128) for x in (r_ap, r_an, r_apc, r_anc)]
    cT = [x.reshape(NB, 8, BM // 128, 128).transpose(0, 2, 1, 3).reshape(N // 128, 8, 128)
          for x in (c_ap, c_an, c_apc, c_anc)]

    loss = pl.pallas_call(
        _finish_kernel,
        in_specs=[pl.BlockSpec((N // 128, 128), lambda: (0, 0))] * 4
        + [pl.BlockSpec((N // 128, 8, 128), lambda: (0, 0, 0))] * 4,
        out_specs=pl.BlockSpec((1, 1), lambda: (0, 0)),
        out_shape=jax.ShapeDtypeStruct((1, 1), jnp.float32),
    )(*rT, *cT)
    return loss.reshape(())
